# Initial kernel scaffold; baseline (speedup 1.0000x reference)
import jax
import jax.numpy as jnp
from jax.experimental import pallas as pl


def kernel(model_prediction, query, keys_ds, vals_ds, W1, b1, W2, b2):
    raise NotImplementedError("write your pallas kernel here")


# speedup vs baseline: 6.2630x; 6.2630x over previous
import functools

import jax
import jax.numpy as jnp
from jax import lax
from jax.experimental import pallas as pl
from jax.experimental.pallas import tpu as pltpu
from jax.experimental.pallas import tpu_sc as plsc

K = 32
HID = 32
N_DS = 100000
D_FEAT = 128
Q = 1024

GRP = 128
N_PAD = 100352
G = N_PAD // GRP
G_PAD = 896
QB = 256
NB = 2048
NT = N_PAD // NB

BIG = 1e30
IBIG = 2**30


def _s0_body(k_ref, ksq_ref):
    k = k_ref[...]
    ksq_ref[...] = jnp.sum(k * k, axis=1, keepdims=True)


def _s1_body(q_ref, k_ref, ksq_ref, dist_ref, gmin_ref):
    ni = pl.program_id(1)
    q = q_ref[...]
    k = k_ref[...]
    qk = lax.dot_general(q, k, (((1,), (1,)), ((), ())),
                         preferred_element_type=jnp.float32)
    q_sq = jnp.sum(q * q, axis=1, keepdims=True)
    dist = q_sq - 2.0 * qk + ksq_ref[...]
    col = ni * NB + lax.broadcasted_iota(jnp.int32, (QB, NB), 1)
    dist = jnp.where(col >= N_DS, BIG, dist)
    dist_ref[...] = dist
    g = jnp.min(dist.reshape(QB, NB // GRP, GRP), axis=2)
    gmin_ref[...] = g.reshape(QB, 1, 1, NB // GRP)


def _s2_body(gmin_ref, gids_ref, idxa_ref):
    qi = pl.program_id(0)
    lane = lax.broadcasted_iota(jnp.int32, (QB, G_PAD), 1)
    kiota = lax.broadcasted_iota(jnp.int32, (QB, K), 1)

    def step(k, carry):
        g, gacc = carry
        m = jnp.min(g, axis=1, keepdims=True)
        gid = jnp.min(jnp.where(g == m, lane, IBIG), axis=1, keepdims=True)
        gacc = jnp.where(kiota == k, gid, gacc)
        g = jnp.where(lane == gid, BIG, g)
        return g, gacc

    g0 = gmin_ref[...]
    _, gids = lax.fori_loop(0, K, step, (g0, jnp.zeros((QB, K), jnp.int32)))
    row = qi * QB + lax.broadcasted_iota(jnp.int32, (QB, K), 0)
    gids_ref[...] = gids
    idxa_ref[...] = row * G + gids


def _s4_body(cd_ref, cv_ref, gids_ref, mp_ref, w1_ref, b1_ref, w2_ref,
             b2_ref, out_ref):
    cd = cd_ref[...]
    cv = cv_ref[...]
    gids = gids_ref[...]
    off = lax.broadcasted_iota(jnp.int32, (QB, K, GRP), 2)
    gidx = (gids[:, :, None] * GRP + off).reshape(QB, K * GRP)
    kiota = lax.broadcasted_iota(jnp.int32, (QB, K), 1)

    def step(k, carry):
        cdc, dacc, vacc = carry
        m = jnp.min(cdc, axis=1, keepdims=True)
        sel = jnp.min(jnp.where(cdc == m, gidx, IBIG), axis=1, keepdims=True)
        hit = gidx == sel
        v = jnp.sum(jnp.where(hit, cv, 0.0), axis=1, keepdims=True)
        dacc = jnp.where(kiota == k, m, dacc)
        vacc = jnp.where(kiota == k, v, vacc)
        cdc = jnp.where(hit, BIG, cdc)
        return cdc, dacc, vacc

    zk = jnp.zeros((QB, K), jnp.float32)
    _, dmat, vmat = lax.fori_loop(0, K, step, (cd, zk, zk))
    net_in = jnp.concatenate([dmat, vmat], axis=1)
    h = lax.dot_general(net_in, w1_ref[...], (((1,), (0,)), ((), ())),
                        preferred_element_type=jnp.float32) + b1_ref[...]
    h = jnp.maximum(h, 0.0)
    logits = lax.dot_general(h, w2_ref[...], (((1,), (0,)), ((), ())),
                             preferred_element_type=jnp.float32) + b2_ref[...]
    mx = jnp.max(logits, axis=1, keepdims=True)
    e = jnp.exp(logits - mx)
    p = e / jnp.sum(e, axis=1, keepdims=True)
    knn_v = jnp.concatenate([mp_ref[...], vmat], axis=1)
    out_ref[...] = jnp.sum(p * knn_v, axis=1, keepdims=True)


def _sc_gather(dist_tbl, vals_tbl, idxa, gidx):
    info = plsc.get_sparse_core_info()
    nc, ns = info.num_cores, info.num_subcores
    nw = nc * ns
    b_tot = Q * K
    b_per_w = b_tot // nw
    ch = 128
    n_ch = b_per_w // ch
    mesh = plsc.VectorSubcoreMesh(core_axis_name="c", subcore_axis_name="s")

    @functools.partial(
        pl.kernel, mesh=mesh,
        out_type=[jax.ShapeDtypeStruct((b_tot, GRP), jnp.float32),
                  jax.ShapeDtypeStruct((b_tot, GRP), jnp.float32)],
        scratch_types=[
            pltpu.VMEM((ch,), jnp.int32),
            pltpu.VMEM((ch,), jnp.int32),
            pltpu.VMEM((ch, GRP), jnp.float32),
            pltpu.VMEM((ch, GRP), jnp.float32),
            pltpu.SemaphoreType.DMA,
            pltpu.SemaphoreType.DMA,
        ],
    )
    def k(dtbl, vtbl, ia, ib, outd, outv, ia_v, ib_v, rd_v, rv_v, semd, semv):
        wid = lax.axis_index("s") * nc + lax.axis_index("c")
        base = wid * b_per_w
        for c in range(n_ch):
            off = base + c * ch
            pltpu.sync_copy(ia.at[pl.ds(off, ch)], ia_v)
            pltpu.sync_copy(ib.at[pl.ds(off, ch)], ib_v)
            cp_d = pltpu.async_copy(dtbl.at[ia_v], rd_v, semd)
            cp_v = pltpu.async_copy(vtbl.at[ib_v], rv_v, semv)
            cp_d.wait()
            cp_v.wait()
            pltpu.sync_copy(rd_v, outd.at[pl.ds(off, ch)])
            pltpu.sync_copy(rv_v, outv.at[pl.ds(off, ch)])

    return k(dist_tbl, vals_tbl, idxa, gidx)


def kernel(model_prediction, query, keys_ds, vals_ds, W1, b1, W2, b2):
    keys_pad = jnp.pad(keys_ds, ((0, N_PAD - N_DS), (0, 0)))
    vals_pad = jnp.pad(vals_ds, (0, N_PAD - N_DS))

    ksq_col = pl.pallas_call(
        _s0_body,
        grid=(NT,),
        in_specs=[pl.BlockSpec((NB, D_FEAT), lambda ni: (ni, 0))],
        out_specs=pl.BlockSpec((NB, 1), lambda ni: (ni, 0)),
        out_shape=jax.ShapeDtypeStruct((N_PAD, 1), jnp.float32),
    )(keys_pad)
    ksq_row = ksq_col.reshape(1, N_PAD)

    dists, gmin4 = pl.pallas_call(
        _s1_body,
        grid=(Q // QB, NT),
        in_specs=[
            pl.BlockSpec((QB, D_FEAT), lambda qi, ni: (qi, 0)),
            pl.BlockSpec((NB, D_FEAT), lambda qi, ni: (ni, 0)),
            pl.BlockSpec((1, NB), lambda qi, ni: (0, ni)),
        ],
        out_specs=[
            pl.BlockSpec((QB, NB), lambda qi, ni: (qi, ni)),
            pl.BlockSpec((QB, 1, 1, NB // GRP), lambda qi, ni: (qi, ni, 0, 0)),
        ],
        out_shape=[
            jax.ShapeDtypeStruct((Q, N_PAD), jnp.float32),
            jax.ShapeDtypeStruct((Q, NT, 1, NB // GRP), jnp.float32),
        ],
    )(query, keys_pad, ksq_row)

    gmin = jnp.pad(gmin4.reshape(Q, G), ((0, 0), (0, G_PAD - G)),
                   constant_values=BIG)

    gids, idxa = pl.pallas_call(
        _s2_body,
        grid=(Q // QB,),
        in_specs=[pl.BlockSpec((QB, G_PAD), lambda qi: (qi, 0))],
        out_specs=[pl.BlockSpec((QB, K), lambda qi: (qi, 0)),
                   pl.BlockSpec((QB, K), lambda qi: (qi, 0))],
        out_shape=[jax.ShapeDtypeStruct((Q, K), jnp.int32),
                   jax.ShapeDtypeStruct((Q, K), jnp.int32)],
    )(gmin)

    cand_d, cand_v = _sc_gather(dists.reshape(Q * G, GRP),
                                vals_pad.reshape(G, GRP),
                                idxa.reshape(-1), gids.reshape(-1))

    out = pl.pallas_call(
        _s4_body,
        grid=(Q // QB,),
        in_specs=[
            pl.BlockSpec((QB, K * GRP), lambda qi: (qi, 0)),
            pl.BlockSpec((QB, K * GRP), lambda qi: (qi, 0)),
            pl.BlockSpec((QB, K), lambda qi: (qi, 0)),
            pl.BlockSpec((QB, 1), lambda qi: (qi, 0)),
            pl.BlockSpec((2 * K, HID), lambda qi: (0, 0)),
            pl.BlockSpec((1, HID), lambda qi: (0, 0)),
            pl.BlockSpec((HID, 1 + K), lambda qi: (0, 0)),
            pl.BlockSpec((1, 1 + K), lambda qi: (0, 0)),
        ],
        out_specs=pl.BlockSpec((QB, 1), lambda qi: (qi, 0)),
        out_shape=jax.ShapeDtypeStruct((Q, 1), jnp.float32),
    )(cand_d.reshape(Q, K * GRP), cand_v.reshape(Q, K * GRP), gids,
      model_prediction, W1, b1.reshape(1, HID), W2, b2.reshape(1, 1 + K))

    return out.reshape(Q)


# speedup vs baseline: 6.6137x; 1.0560x over previous
import functools

import jax
import jax.numpy as jnp
from jax import lax
from jax.experimental import pallas as pl
from jax.experimental.pallas import tpu as pltpu
from jax.experimental.pallas import tpu_sc as plsc

K = 32
HID = 32
N_DS = 100000
D_FEAT = 128
Q = 1024

GRP = 128
N_PAD = 100352
G = N_PAD // GRP
G_PAD = 896
QB = 256
NB = 2048
NT = N_PAD // NB

BIG = 1e30
IBIG = 2**30


def _s1_body(q_ref, k_ref, ksq_ref, dist_ref, gmin_ref):
    q = q_ref[...]
    k = k_ref[...]
    qk = lax.dot_general(q, k, (((1,), (1,)), ((), ())),
                         preferred_element_type=jnp.float32)
    q_sq = jnp.sum(q * q, axis=1, keepdims=True)
    dist = q_sq - 2.0 * qk + ksq_ref[...]
    dist_ref[...] = dist
    g = jnp.min(dist.reshape(QB, NB // GRP, GRP), axis=2)
    gmin_ref[...] = g.reshape(QB, 1, 1, NB // GRP)


def _s2_body(gmin_ref, gids_ref, idxa_ref):
    qi = pl.program_id(0)
    lane = lax.broadcasted_iota(jnp.int32, (QB, G_PAD), 1)
    kiota = lax.broadcasted_iota(jnp.int32, (QB, K), 1)

    def step(k, carry):
        g, gacc = carry
        m = jnp.min(g, axis=1, keepdims=True)
        gid = jnp.min(jnp.where(g == m, lane, IBIG), axis=1, keepdims=True)
        gacc = jnp.where(kiota == k, gid, gacc)
        g = jnp.where(lane == gid, BIG, g)
        return g, gacc

    g0 = gmin_ref[...]
    _, gids = lax.fori_loop(0, K, step, (g0, jnp.zeros((QB, K), jnp.int32)))
    row = qi * QB + lax.broadcasted_iota(jnp.int32, (QB, K), 0)
    gids_ref[...] = gids
    idxa_ref[...] = row * G + gids


def _s4_body(cd_ref, cv_ref, gids_ref, mp_ref, w1_ref, b1_ref, w2_ref,
             b2_ref, out_ref):
    cd = cd_ref[...]
    cv = cv_ref[...]
    gids = gids_ref[...]
    off = lax.broadcasted_iota(jnp.int32, (QB, K, GRP), 2)
    gidx = (gids[:, :, None] * GRP + off).reshape(QB, K * GRP)
    kiota = lax.broadcasted_iota(jnp.int32, (QB, K), 1)

    def step(k, carry):
        cdc, dacc, vacc = carry
        m = jnp.min(cdc, axis=1, keepdims=True)
        sel = jnp.min(jnp.where(cdc == m, gidx, IBIG), axis=1, keepdims=True)
        hit = gidx == sel
        v = jnp.sum(jnp.where(hit, cv, 0.0), axis=1, keepdims=True)
        dacc = jnp.where(kiota == k, m, dacc)
        vacc = jnp.where(kiota == k, v, vacc)
        cdc = jnp.where(hit, BIG, cdc)
        return cdc, dacc, vacc

    zk = jnp.zeros((QB, K), jnp.float32)
    _, dmat, vmat = lax.fori_loop(0, K, step, (cd, zk, zk))
    net_in = jnp.concatenate([dmat, vmat], axis=1)
    h = lax.dot_general(net_in, w1_ref[...], (((1,), (0,)), ((), ())),
                        preferred_element_type=jnp.float32) + b1_ref[...]
    h = jnp.maximum(h, 0.0)
    logits = lax.dot_general(h, w2_ref[...], (((1,), (0,)), ((), ())),
                             preferred_element_type=jnp.float32) + b2_ref[...]
    mx = jnp.max(logits, axis=1, keepdims=True)
    e = jnp.exp(logits - mx)
    p = e / jnp.sum(e, axis=1, keepdims=True)
    knn_v = jnp.concatenate([mp_ref[...], vmat], axis=1)
    out_ref[...] = jnp.sum(p * knn_v, axis=1, keepdims=True)


def _sc_gather(dist_tbl, vals_tbl, idxa, gidx):
    info = plsc.get_sparse_core_info()
    nc, ns = info.num_cores, info.num_subcores
    nw = nc * ns
    b_tot = Q * K
    b_per_w = b_tot // nw
    ch = 128
    n_ch = b_per_w // ch
    mesh = plsc.VectorSubcoreMesh(core_axis_name="c", subcore_axis_name="s")

    @functools.partial(
        pl.kernel, mesh=mesh,
        out_type=[jax.ShapeDtypeStruct((b_tot, GRP), jnp.float32),
                  jax.ShapeDtypeStruct((b_tot, GRP), jnp.float32)],
        scratch_types=[
            pltpu.VMEM((n_ch, ch), jnp.int32),
            pltpu.VMEM((n_ch, ch), jnp.int32),
            pltpu.VMEM((2, ch, GRP), jnp.float32),
            pltpu.VMEM((2, ch, GRP), jnp.float32),
            pltpu.SemaphoreType.DMA,
            pltpu.SemaphoreType.DMA,
            pltpu.SemaphoreType.DMA,
            pltpu.SemaphoreType.DMA,
            pltpu.SemaphoreType.DMA,
            pltpu.SemaphoreType.DMA,
            pltpu.SemaphoreType.DMA,
            pltpu.SemaphoreType.DMA,
        ],
    )
    def k(dtbl, vtbl, ia, ib, outd, outv, ia_v, ib_v, rd_v, rv_v,
          semd0, semd1, semv0, semv1, semwd0, semwd1, semwv0, semwv1):
        wid = lax.axis_index("s") * nc + lax.axis_index("c")
        base = wid * b_per_w
        pltpu.sync_copy(ia.at[pl.ds(wid * n_ch, n_ch)], ia_v)
        pltpu.sync_copy(ib.at[pl.ds(wid * n_ch, n_ch)], ib_v)
        semd = (semd0, semd1)
        semv = (semv0, semv1)
        semwd = (semwd0, semwd1)
        semwv = (semwv0, semwv1)
        gathers, writes = {}, {}

        def start(c):
            buf = c % 2
            gathers[c] = (
                pltpu.async_copy(dtbl.at[ia_v.at[c]], rd_v.at[buf], semd[buf]),
                pltpu.async_copy(vtbl.at[ib_v.at[c]], rv_v.at[buf], semv[buf]),
            )

        start(0)
        start(1)
        for c in range(n_ch):
            buf = c % 2
            cp_d, cp_v = gathers.pop(c)
            cp_d.wait()
            cp_v.wait()
            writes[c] = (
                pltpu.async_copy(rd_v.at[buf], outd.at[pl.ds(base + c * ch, ch)],
                                 semwd[buf]),
                pltpu.async_copy(rv_v.at[buf], outv.at[pl.ds(base + c * ch, ch)],
                                 semwv[buf]),
            )
            if c + 2 < n_ch:
                wd, wv = writes.pop(c)
                wd.wait()
                wv.wait()
                start(c + 2)
        for c in list(writes):
            wd, wv = writes.pop(c)
            wd.wait()
            wv.wait()

    return k(dist_tbl, vals_tbl, idxa.reshape(-1, ch), gidx.reshape(-1, ch))


def kernel(model_prediction, query, keys_ds, vals_ds, W1, b1, W2, b2):
    keys_pad = jnp.pad(keys_ds, ((0, N_PAD - N_DS), (0, 0)))
    vals_pad = jnp.pad(vals_ds, (0, N_PAD - N_DS))

    ksq = jnp.sum(keys_ds * keys_ds, axis=-1)
    ksq_row = jnp.concatenate(
        [ksq, jnp.full((N_PAD - N_DS,), BIG, jnp.float32)]).reshape(1, N_PAD)

    dists, gmin4 = pl.pallas_call(
        _s1_body,
        grid=(Q // QB, NT),
        in_specs=[
            pl.BlockSpec((QB, D_FEAT), lambda qi, ni: (qi, 0)),
            pl.BlockSpec((NB, D_FEAT), lambda qi, ni: (ni, 0)),
            pl.BlockSpec((1, NB), lambda qi, ni: (0, ni)),
        ],
        out_specs=[
            pl.BlockSpec((QB, NB), lambda qi, ni: (qi, ni)),
            pl.BlockSpec((QB, 1, 1, NB // GRP), lambda qi, ni: (qi, ni, 0, 0)),
        ],
        out_shape=[
            jax.ShapeDtypeStruct((Q, N_PAD), jnp.float32),
            jax.ShapeDtypeStruct((Q, NT, 1, NB // GRP), jnp.float32),
        ],
    )(query, keys_pad, ksq_row)

    gmin = jnp.pad(gmin4.reshape(Q, G), ((0, 0), (0, G_PAD - G)),
                   constant_values=BIG)

    gids, idxa = pl.pallas_call(
        _s2_body,
        grid=(Q // QB,),
        in_specs=[pl.BlockSpec((QB, G_PAD), lambda qi: (qi, 0))],
        out_specs=[pl.BlockSpec((QB, K), lambda qi: (qi, 0)),
                   pl.BlockSpec((QB, K), lambda qi: (qi, 0))],
        out_shape=[jax.ShapeDtypeStruct((Q, K), jnp.int32),
                   jax.ShapeDtypeStruct((Q, K), jnp.int32)],
    )(gmin)

    cand_d, cand_v = _sc_gather(dists.reshape(Q * G, GRP),
                                vals_pad.reshape(G, GRP), idxa, gids)

    out = pl.pallas_call(
        _s4_body,
        grid=(Q // QB,),
        in_specs=[
            pl.BlockSpec((QB, K * GRP), lambda qi: (qi, 0)),
            pl.BlockSpec((QB, K * GRP), lambda qi: (qi, 0)),
            pl.BlockSpec((QB, K), lambda qi: (qi, 0)),
            pl.BlockSpec((QB, 1), lambda qi: (qi, 0)),
            pl.BlockSpec((2 * K, HID), lambda qi: (0, 0)),
            pl.BlockSpec((1, HID), lambda qi: (0, 0)),
            pl.BlockSpec((HID, 1 + K), lambda qi: (0, 0)),
            pl.BlockSpec((1, 1 + K), lambda qi: (0, 0)),
        ],
        out_specs=pl.BlockSpec((QB, 1), lambda qi: (qi, 0)),
        out_shape=jax.ShapeDtypeStruct((Q, 1), jnp.float32),
    )(cand_d.reshape(Q, K * GRP), cand_v.reshape(Q, K * GRP), gids,
      model_prediction, W1, b1.reshape(1, HID), W2, b2.reshape(1, 1 + K))

    return out.reshape(Q)


# speedup vs baseline: 7.3665x; 1.1138x over previous
import functools

import jax
import jax.numpy as jnp
from jax import lax
from jax.experimental import pallas as pl
from jax.experimental.pallas import tpu as pltpu
from jax.experimental.pallas import tpu_sc as plsc

K = 32
HID = 32
N_DS = 100000
D_FEAT = 128
Q = 1024

GRP = 128
N_PAD = 100352
G = N_PAD // GRP
G_PAD = 896
QB = 256
QB1 = 1024
NB = 2048
NT = N_PAD // NB

BIG = 1e30
IBIG = 2**30


def _s1_body(q_ref, k_ref, ksq_ref, dist_ref, gmin_ref):
    q = q_ref[...]
    k = k_ref[...]
    qk2 = lax.dot_general(q * -2.0, k, (((1,), (1,)), ((), ())),
                          preferred_element_type=jnp.float32)
    q_sq = jnp.sum(q * q, axis=1, keepdims=True)
    dist = (q_sq + qk2) + ksq_ref[...]
    dist_ref[...] = dist
    g = jnp.min(dist.reshape(QB1, NB // GRP, GRP), axis=2)
    gmin_ref[...] = g.reshape(QB1, 1, 1, NB // GRP)


def _s2_body(gmin_ref, gids_ref, idxa_ref):
    qi = pl.program_id(0)
    lane = lax.broadcasted_iota(jnp.int32, (QB, G_PAD), 1)
    kiota = lax.broadcasted_iota(jnp.int32, (QB, K), 1)

    def step(k, carry):
        g, gacc = carry
        m = jnp.min(g, axis=1, keepdims=True)
        gid = jnp.min(jnp.where(g == m, lane, IBIG), axis=1, keepdims=True)
        gacc = jnp.where(kiota == k, gid, gacc)
        g = jnp.where(lane == gid, BIG, g)
        return g, gacc

    g0 = gmin_ref[...]
    _, gids = lax.fori_loop(0, K, step, (g0, jnp.zeros((QB, K), jnp.int32)))
    row = qi * QB + lax.broadcasted_iota(jnp.int32, (QB, K), 0)
    gids_ref[...] = gids
    idxa_ref[...] = row * G + gids


def _s4_body(cd_ref, cv_ref, gids_ref, mp_ref, w1_ref, b1_ref, w2_ref,
             b2_ref, out_ref):
    cd = cd_ref[...]
    cv = cv_ref[...]
    gids = gids_ref[...]
    off = lax.broadcasted_iota(jnp.int32, (QB, K, GRP), 2)
    gidx = (gids[:, :, None] * GRP + off).reshape(QB, K * GRP)
    kiota = lax.broadcasted_iota(jnp.int32, (QB, K), 1)

    def step(k, carry):
        cdc, dacc, vacc = carry
        m = jnp.min(cdc, axis=1, keepdims=True)
        sel = jnp.min(jnp.where(cdc == m, gidx, IBIG), axis=1, keepdims=True)
        hit = gidx == sel
        v = jnp.sum(jnp.where(hit, cv, 0.0), axis=1, keepdims=True)
        dacc = jnp.where(kiota == k, m, dacc)
        vacc = jnp.where(kiota == k, v, vacc)
        cdc = jnp.where(hit, BIG, cdc)
        return cdc, dacc, vacc

    zk = jnp.zeros((QB, K), jnp.float32)
    _, dmat, vmat = lax.fori_loop(0, K, step, (cd, zk, zk))
    net_in = jnp.concatenate([dmat, vmat], axis=1)
    h = lax.dot_general(net_in, w1_ref[...], (((1,), (0,)), ((), ())),
                        preferred_element_type=jnp.float32) + b1_ref[...]
    h = jnp.maximum(h, 0.0)
    logits = lax.dot_general(h, w2_ref[...], (((1,), (0,)), ((), ())),
                             preferred_element_type=jnp.float32) + b2_ref[...]
    mx = jnp.max(logits, axis=1, keepdims=True)
    e = jnp.exp(logits - mx)
    p = e / jnp.sum(e, axis=1, keepdims=True)
    knn_v = jnp.concatenate([mp_ref[...], vmat], axis=1)
    out_ref[...] = jnp.sum(p * knn_v, axis=1, keepdims=True)


def _sc_gather(dist_tbl, vals_tbl, idxa, gidx):
    info = plsc.get_sparse_core_info()
    nc, ns = info.num_cores, info.num_subcores
    nw = nc * ns
    b_tot = Q * K
    b_per_w = b_tot // nw
    ch = 128
    n_ch = b_per_w // ch
    mesh = plsc.VectorSubcoreMesh(core_axis_name="c", subcore_axis_name="s")

    @functools.partial(
        pl.kernel, mesh=mesh,
        out_type=[jax.ShapeDtypeStruct((b_tot, GRP), jnp.float32),
                  jax.ShapeDtypeStruct((b_tot, GRP), jnp.float32)],
        scratch_types=[
            pltpu.VMEM((n_ch, ch), jnp.int32),
            pltpu.VMEM((n_ch, ch), jnp.int32),
            pltpu.VMEM((2, ch, GRP), jnp.float32),
            pltpu.VMEM((2, ch, GRP), jnp.float32),
            pltpu.SemaphoreType.DMA,
            pltpu.SemaphoreType.DMA,
            pltpu.SemaphoreType.DMA,
            pltpu.SemaphoreType.DMA,
            pltpu.SemaphoreType.DMA,
            pltpu.SemaphoreType.DMA,
            pltpu.SemaphoreType.DMA,
            pltpu.SemaphoreType.DMA,
        ],
    )
    def k(dtbl, vtbl, ia, ib, outd, outv, ia_v, ib_v, rd_v, rv_v,
          semd0, semd1, semv0, semv1, semwd0, semwd1, semwv0, semwv1):
        wid = lax.axis_index("s") * nc + lax.axis_index("c")
        base = wid * b_per_w
        pltpu.sync_copy(ia.at[pl.ds(wid * n_ch, n_ch)], ia_v)
        pltpu.sync_copy(ib.at[pl.ds(wid * n_ch, n_ch)], ib_v)
        semd = (semd0, semd1)
        semv = (semv0, semv1)
        semwd = (semwd0, semwd1)
        semwv = (semwv0, semwv1)
        gathers, writes = {}, {}

        def start(c):
            buf = c % 2
            gathers[c] = (
                pltpu.async_copy(dtbl.at[ia_v.at[c]], rd_v.at[buf], semd[buf]),
                pltpu.async_copy(vtbl.at[ib_v.at[c]], rv_v.at[buf], semv[buf]),
            )

        start(0)
        start(1)
        for c in range(n_ch):
            buf = c % 2
            cp_d, cp_v = gathers.pop(c)
            cp_d.wait()
            cp_v.wait()
            writes[c] = (
                pltpu.async_copy(rd_v.at[buf], outd.at[pl.ds(base + c * ch, ch)],
                                 semwd[buf]),
                pltpu.async_copy(rv_v.at[buf], outv.at[pl.ds(base + c * ch, ch)],
                                 semwv[buf]),
            )
            if c + 2 < n_ch:
                wd, wv = writes.pop(c)
                wd.wait()
                wv.wait()
                start(c + 2)
        for c in list(writes):
            wd, wv = writes.pop(c)
            wd.wait()
            wv.wait()

    return k(dist_tbl, vals_tbl, idxa.reshape(-1, ch), gidx.reshape(-1, ch))


def kernel(model_prediction, query, keys_ds, vals_ds, W1, b1, W2, b2):
    keys_pad = jnp.pad(keys_ds, ((0, N_PAD - N_DS), (0, 0)))
    vals_pad = jnp.pad(vals_ds, (0, N_PAD - N_DS))

    ksq = jnp.sum(keys_ds * keys_ds, axis=-1)
    ksq_row = jnp.concatenate(
        [ksq, jnp.full((N_PAD - N_DS,), BIG, jnp.float32)]).reshape(1, N_PAD)

    dists, gmin4 = pl.pallas_call(
        _s1_body,
        grid=(Q // QB1, NT),
        in_specs=[
            pl.BlockSpec((QB1, D_FEAT), lambda qi, ni: (qi, 0)),
            pl.BlockSpec((NB, D_FEAT), lambda qi, ni: (ni, 0)),
            pl.BlockSpec((1, NB), lambda qi, ni: (0, ni)),
        ],
        out_specs=[
            pl.BlockSpec((QB1, NB), lambda qi, ni: (qi, ni)),
            pl.BlockSpec((QB1, 1, 1, NB // GRP), lambda qi, ni: (qi, ni, 0, 0)),
        ],
        out_shape=[
            jax.ShapeDtypeStruct((Q, N_PAD), jnp.float32),
            jax.ShapeDtypeStruct((Q, NT, 1, NB // GRP), jnp.float32),
        ],
    )(query, keys_pad, ksq_row)

    gmin = jnp.pad(gmin4.reshape(Q, G), ((0, 0), (0, G_PAD - G)),
                   constant_values=BIG)

    gids, idxa = pl.pallas_call(
        _s2_body,
        grid=(Q // QB,),
        in_specs=[pl.BlockSpec((QB, G_PAD), lambda qi: (qi, 0))],
        out_specs=[pl.BlockSpec((QB, K), lambda qi: (qi, 0)),
                   pl.BlockSpec((QB, K), lambda qi: (qi, 0))],
        out_shape=[jax.ShapeDtypeStruct((Q, K), jnp.int32),
                   jax.ShapeDtypeStruct((Q, K), jnp.int32)],
    )(gmin)

    cand_d, cand_v = _sc_gather(dists.reshape(Q * G, GRP),
                                vals_pad.reshape(G, GRP), idxa, gids)

    out = pl.pallas_call(
        _s4_body,
        grid=(Q // QB,),
        in_specs=[
            pl.BlockSpec((QB, K * GRP), lambda qi: (qi, 0)),
            pl.BlockSpec((QB, K * GRP), lambda qi: (qi, 0)),
            pl.BlockSpec((QB, K), lambda qi: (qi, 0)),
            pl.BlockSpec((QB, 1), lambda qi: (qi, 0)),
            pl.BlockSpec((2 * K, HID), lambda qi: (0, 0)),
            pl.BlockSpec((1, HID), lambda qi: (0, 0)),
            pl.BlockSpec((HID, 1 + K), lambda qi: (0, 0)),
            pl.BlockSpec((1, 1 + K), lambda qi: (0, 0)),
        ],
        out_specs=pl.BlockSpec((QB, 1), lambda qi: (qi, 0)),
        out_shape=jax.ShapeDtypeStruct((Q, 1), jnp.float32),
    )(cand_d.reshape(Q, K * GRP), cand_v.reshape(Q, K * GRP), gids,
      model_prediction, W1, b1.reshape(1, HID), W2, b2.reshape(1, 1 + K))

    return out.reshape(Q)
